# baseline (speedup 1.0000x reference)
import functools
import math

import jax
import jax.numpy as jnp
from jax import lax
from jax.experimental import pallas as pl
from jax.experimental.pallas import tpu as pltpu
from jax.experimental.pallas import tpu_sc as plsc

D_EMBED = 64
VOCAB = 1000
SCALE = math.sqrt(float(D_EMBED))

NUM_CORES = 2
NUM_SUBCORES = 16
NUM_WORKERS = NUM_CORES * NUM_SUBCORES


def _scale_table_body(t_ref, o_ref):
    o_ref[...] = t_ref[...] * SCALE


@jax.jit
def _scale_table(table):
    return pl.pallas_call(
        _scale_table_body,
        out_shape=jax.ShapeDtypeStruct(table.shape, table.dtype),
    )(table)


def _make_gather(total, chunk, n_buf):
    assert total % (NUM_WORKERS * chunk * n_buf) == 0
    per_worker = total // NUM_WORKERS
    n_chunks = per_worker // chunk
    n_groups = n_chunks // n_buf
    mesh = plsc.VectorSubcoreMesh(
        core_axis_name="c", subcore_axis_name="s",
        num_cores=NUM_CORES, num_subcores=NUM_SUBCORES,
    )

    @functools.partial(
        pl.kernel,
        out_type=jax.ShapeDtypeStruct((total, D_EMBED), jnp.float32),
        mesh=mesh,
        scratch_types=[
            pltpu.VMEM((per_worker,), jnp.int32),
            pltpu.VMEM((n_buf, chunk, D_EMBED), jnp.float32),
            pltpu.SemaphoreType.DMA((n_buf,)),
            pltpu.SemaphoreType.DMA((n_buf,)),
        ],
        compiler_params=pltpu.CompilerParams(use_tc_tiling_on_sc=False),
    )
    def gather_kernel(idx_hbm, tbl_hbm, out_hbm, idx_v, rows_v, gsem, ssem):
        wid = lax.axis_index("s") * NUM_CORES + lax.axis_index("c")
        base = wid * per_worker
        pltpu.sync_copy(idx_hbm.at[pl.ds(base, per_worker)], idx_v)

        def gather_start(i, b):
            pltpu.async_copy(
                tbl_hbm.at[idx_v.at[pl.ds(i * chunk, chunk)]],
                rows_v.at[b], gsem.at[b])

        def store_start(i, b):
            pltpu.async_copy(
                rows_v.at[b], out_hbm.at[pl.ds(base + i * chunk, chunk)],
                ssem.at[b])

        def gather_wait(i, b):
            pltpu.make_async_copy(
                tbl_hbm.at[idx_v.at[pl.ds(i * chunk, chunk)]],
                rows_v.at[b], gsem.at[b]).wait()

        def store_wait(i, b):
            pltpu.make_async_copy(
                rows_v.at[b], out_hbm.at[pl.ds(base + i * chunk, chunk)],
                ssem.at[b]).wait()

        for b in range(n_buf):
            gather_start(b, b)

        def group(g, carry):
            i0 = g * n_buf
            for b in range(n_buf):
                gather_wait(i0 + b, b)
                store_start(i0 + b, b)
            for b in range(n_buf):
                store_wait(i0 + b, b)
                gather_start(i0 + n_buf + b, b)
            return carry

        lax.fori_loop(0, n_groups - 1, group, 0)

        i0 = (n_groups - 1) * n_buf
        for b in range(n_buf):
            gather_wait(i0 + b, b)
            store_start(i0 + b, b)
        for b in range(n_buf):
            store_wait(i0 + b, b)

    return gather_kernel


_gather = _make_gather(4096 * 200, 320, 4)


@jax.jit
def kernel(x, table):
    scaled = _scale_table(table)
    flat = x.reshape(-1)
    out = _gather(flat, scaled)
    return out.reshape(x.shape + (D_EMBED,))
